# baseline (speedup 1.0000x reference)
import functools

import jax
import jax.numpy as jnp
from jax import lax
from jax.experimental import pallas as pl
from jax.experimental.pallas import tpu as pltpu
from jax.experimental.pallas import tpu_sc as plsc

_EMB_DIM = 128
_NUM_SEG = 26
_BATCH = 16384
_ROW = _NUM_SEG * _EMB_DIM
_CHUNK_ROWS = 64
_NBUF = 8


def _permute_sc(pooled_embs):
    info = plsc.get_sparse_core_info()
    num_workers = info.num_cores * info.num_subcores
    rows_per_w = _BATCH // num_workers
    n_rchunks = rows_per_w // _CHUNK_ROWS
    assert n_rchunks == _NBUF
    mesh = plsc.VectorSubcoreMesh(core_axis_name="c", subcore_axis_name="s")

    @functools.partial(
        pl.kernel,
        mesh=mesh,
        out_type=jax.ShapeDtypeStruct((_BATCH, _ROW), jnp.float32),
        scratch_types=(
            [pltpu.VMEM((_CHUNK_ROWS, _EMB_DIM), jnp.float32)] * _NBUF
            + [pltpu.SemaphoreType.DMA] * (2 * _NBUF)
        ),
    )
    def k(in_hbm, out_hbm, *scr):
        bufs = scr[:_NBUF]
        gsems = scr[_NBUF : 2 * _NBUF]
        ssems = scr[2 * _NBUF :]
        wid = lax.axis_index("s") * info.num_cores + lax.axis_index("c")
        row_base = wid * rows_per_w
        stag = lax.rem(wid, _NUM_SEG)

        def rot(j):
            jr = j + stag
            return jnp.where(jr >= _NUM_SEG, jr - _NUM_SEG, jr)

        def gather(j, c, slot):
            src_col = (_NUM_SEG - 1 - rot(j)) * _EMB_DIM
            pltpu.make_async_copy(
                in_hbm.at[
                    pl.ds(row_base + c * _CHUNK_ROWS, _CHUNK_ROWS),
                    pl.ds(src_col, _EMB_DIM),
                ],
                bufs[slot],
                gsems[slot],
            ).start()

        def scatter(j, c, slot):
            pltpu.make_async_copy(
                bufs[slot],
                out_hbm.at[
                    pl.ds(row_base + c * _CHUNK_ROWS, _CHUNK_ROWS),
                    pl.ds(rot(j) * _EMB_DIM, _EMB_DIM),
                ],
                ssems[slot],
            ).start()

        dummy_in = in_hbm.at[pl.ds(0, _CHUNK_ROWS), pl.ds(0, _EMB_DIM)]
        dummy_out = out_hbm.at[pl.ds(0, _CHUNK_ROWS), pl.ds(0, _EMB_DIM)]

        def wait_gather(slot):
            pltpu.make_async_copy(dummy_in, bufs[slot], gsems[slot]).wait()

        def wait_scatter(slot):
            pltpu.make_async_copy(bufs[slot], dummy_out, ssems[slot]).wait()

        for c in range(4):
            gather(0, c, c)
        for t in range(4):
            wait_gather(t)
            scatter(0, t, t)
            gather(0, t + 4, t + 4)

        def body(kk, carry):
            for b in range(8):
                if b < 4:
                    j, c = kk, 4 + b
                else:
                    j, c = kk + 1, b - 4
                slot = (4 + b) % 8
                wait_gather(slot)
                scatter(j, c, slot)
                wait_scatter(b)
                gather(kk + 1, b, b)
            return carry

        lax.fori_loop(0, _NUM_SEG - 1, body, 0)

        for c in range(4, 8):
            wait_gather(c)
            scatter(_NUM_SEG - 1, c, c)
        for slot in range(_NBUF):
            wait_scatter(slot)

    return k(pooled_embs)


def kernel(pooled_embs):
    return _permute_sc(pooled_embs)
